# Initial kernel scaffold; baseline (speedup 1.0000x reference)
import jax
import jax.numpy as jnp
from jax.experimental import pallas as pl


def kernel(points_coords, centers_coords, centers_features, condition, W1, b1, W2, b2):
    raise NotImplementedError("write your pallas kernel here")


# speedup vs baseline: 33.3256x; 33.3256x over previous
import jax
import jax.numpy as jnp
import numpy as np
from jax.experimental import pallas as pl

B, N, M, CIN, C1, C2 = 4, 16384, 1024, 32, 64, 64
BLK = 512


def _fp_body(p_ref, c_ref, f_ref, w1_ref, b1_ref, w2_ref, b2_ref, o_ref):
    p = p_ref[0]
    c = c_ref[0]
    f = f_ref[0]
    pn2 = jnp.sum(p * p, axis=0)
    cm2 = jnp.sum(c * c, axis=0)
    cp = jax.lax.dot_general(c, p, (((0,), (0,)), ((), ())),
                             preferred_element_type=jnp.float32)
    d2 = cm2[:, None] - 2.0 * cp + pn2[None, :]

    iota = jax.lax.broadcasted_iota(jnp.int32, d2.shape, 0)
    inf = jnp.float32(np.inf)
    vals, idxs = [], []
    dcur = d2
    for _ in range(3):
        v = jnp.min(dcur, axis=0)
        i = jnp.min(jnp.where(dcur == v[None, :], iota, M), axis=0)
        vals.append(v)
        idxs.append(i)
        dcur = jnp.where(iota == i[None, :], inf, dcur)

    w = [1.0 / (jnp.sqrt(jnp.maximum(v, 1e-10)) + 1e-8) for v in vals]
    wsum = w[0] + w[1] + w[2]
    w = [wi / wsum for wi in w]

    onehot = jnp.zeros_like(d2)
    for k in range(3):
        onehot = onehot + jnp.where(iota == idxs[k][None, :], w[k][None, :], 0.0)

    xt = jax.lax.dot_general(f, onehot, (((1,), (0,)), ((), ())),
                             preferred_element_type=jnp.float32)
    h1 = jnp.maximum(
        jax.lax.dot_general(w1_ref[...], xt, (((1,), (0,)), ((), ())),
                            preferred_element_type=jnp.float32) + b1_ref[...], 0.0)
    h2 = jnp.maximum(
        jax.lax.dot_general(w2_ref[...], h1, (((1,), (0,)), ((), ())),
                            preferred_element_type=jnp.float32) + b2_ref[...], 0.0)
    o_ref[0] = h2


def kernel(points_coords, centers_coords, centers_features, condition, W1, b1, W2, b2):
    x = pl.pallas_call(
        _fp_body,
        grid=(B, N // BLK),
        in_specs=[
            pl.BlockSpec((1, 3, BLK), lambda b, j: (b, 0, j)),
            pl.BlockSpec((1, 3, M), lambda b, j: (b, 0, 0)),
            pl.BlockSpec((1, CIN, M), lambda b, j: (b, 0, 0)),
            pl.BlockSpec((C1, CIN), lambda b, j: (0, 0)),
            pl.BlockSpec((C1, 1), lambda b, j: (0, 0)),
            pl.BlockSpec((C2, C1), lambda b, j: (0, 0)),
            pl.BlockSpec((C2, 1), lambda b, j: (0, 0)),
        ],
        out_specs=pl.BlockSpec((1, C2, BLK), lambda b, j: (b, 0, j)),
        out_shape=jax.ShapeDtypeStruct((B, C2, N), jnp.float32),
    )(points_coords, centers_coords, centers_features,
      W1, b1.reshape(C1, 1), W2, b2.reshape(C2, 1))
    return (x, points_coords, condition)
